# initial kernel scaffold (unmeasured)
import jax
import jax.numpy as jnp
from jax import lax
from jax.experimental import pallas as pl
from jax.experimental.pallas import tpu as pltpu

N_DEV = 4
N_LAYERS = 3


def kernel(x, Win0, Wout0, Win1, Wout1, Win2, Wout2):
    B, D = x.shape
    rows_per = B // N_DEV

    def body(x_ref, win0_ref, wout0_ref, win1_ref, wout1_ref, win2_ref,
             wout2_ref, out_ref, comm_ref, send_sems, recv_sems):
        my = lax.axis_index("i")

        barrier_sem = pltpu.get_barrier_semaphore()
        for d in range(1, N_DEV):
            pl.semaphore_signal(
                barrier_sem, inc=1,
                device_id=((my + d) % N_DEV,),
                device_id_type=pl.DeviceIdType.MESH,
            )
        pl.semaphore_wait(barrier_sem, N_DEV - 1)

        wins = [win0_ref, win1_ref, win2_ref]
        wouts = [wout0_ref, wout1_ref, wout2_ref]

        x_cur = x_ref[:, :].astype(jnp.bfloat16)
        total = None
        for l in range(N_LAYERS):
            h = jnp.dot(x_cur, wins[l][:, :].astype(jnp.bfloat16),
                        preferred_element_type=jnp.float32)
            h = jnp.maximum(h, 0.0).astype(jnp.bfloat16)
            partial = jnp.dot(h, wouts[l][:, :].astype(jnp.bfloat16),
                              preferred_element_type=jnp.float32)
            comm_ref[l, 0, :, :] = partial
            rdmas = []
            for d in range(1, N_DEV):
                rdma = pltpu.make_async_remote_copy(
                    src_ref=comm_ref.at[l, 0],
                    dst_ref=comm_ref.at[l, d],
                    send_sem=send_sems.at[l, d - 1],
                    recv_sem=recv_sems.at[l, d - 1],
                    device_id=((my + d) % N_DEV,),
                    device_id_type=pl.DeviceIdType.MESH,
                )
                rdma.start()
                rdmas.append(rdma)
            for rdma in rdmas:
                rdma.wait()
            total = (comm_ref[l, 0, :, :] + comm_ref[l, 1, :, :]
                     + comm_ref[l, 2, :, :] + comm_ref[l, 3, :, :])
            x_cur = total.astype(jnp.bfloat16)

        out_ref[:, :] = lax.dynamic_slice(total, (my * rows_per, 0),
                                          (rows_per, D))

    return pl.pallas_call(
        body,
        out_shape=jax.ShapeDtypeStruct((rows_per, D), jnp.float32),
        in_specs=[pl.BlockSpec(memory_space=pltpu.VMEM)] * 7,
        out_specs=pl.BlockSpec(memory_space=pltpu.VMEM),
        scratch_shapes=[
            pltpu.VMEM((N_LAYERS, N_DEV, B, D), jnp.float32),
            pltpu.SemaphoreType.DMA((N_LAYERS, N_DEV - 1)),
            pltpu.SemaphoreType.DMA((N_LAYERS, N_DEV - 1)),
        ],
        compiler_params=pltpu.CompilerParams(collective_id=0),
    )(x, Win0, Wout0, Win1, Wout1, Win2, Wout2)


# baseline (device time: 21260 ns/iter reference)
import jax
import jax.numpy as jnp
from jax import lax
from jax.experimental import pallas as pl
from jax.experimental.pallas import tpu as pltpu

N_DEV = 4
N_LAYERS = 3


def kernel(x, Win0, Wout0, Win1, Wout1, Win2, Wout2):
    B, D = x.shape
    rows_per = B // N_DEV

    def body(x_ref, win0_ref, wout0_ref, win1_ref, wout1_ref, win2_ref,
             wout2_ref, out_ref, comm_ref, send_sems, recv_sems):
        my = lax.axis_index("i")

        barrier_sem = pltpu.get_barrier_semaphore()
        for d in range(1, N_DEV):
            pl.semaphore_signal(
                barrier_sem, inc=1,
                device_id=((my + d) % N_DEV,),
                device_id_type=pl.DeviceIdType.MESH,
            )
        pl.semaphore_wait(barrier_sem, N_DEV - 1)

        wins = [win0_ref, win1_ref, win2_ref]
        wouts = [wout0_ref, wout1_ref, wout2_ref]

        x_cur = x_ref[:, :].astype(jnp.bfloat16)
        total = None
        for l in range(N_LAYERS):
            h = jnp.dot(x_cur, wins[l][:, :].astype(jnp.bfloat16),
                        preferred_element_type=jnp.float32)
            h = jnp.maximum(h, 0.0).astype(jnp.bfloat16)
            partial = jnp.dot(h, wouts[l][:, :].astype(jnp.bfloat16),
                              preferred_element_type=jnp.float32)
            comm_ref[l, 0, :, :] = partial
            rdmas = []
            for d in range(1, N_DEV):
                rdma = pltpu.make_async_remote_copy(
                    src_ref=comm_ref.at[l, 0],
                    dst_ref=comm_ref.at[l, d],
                    send_sem=send_sems.at[l, d - 1],
                    recv_sem=recv_sems.at[l, d - 1],
                    device_id=((my + d) % N_DEV,),
                    device_id_type=pl.DeviceIdType.MESH,
                )
                rdma.start()
                rdmas.append(rdma)
            for rdma in rdmas:
                rdma.wait()
            total = (comm_ref[l, 0, :, :] + comm_ref[l, 1, :, :]
                     + comm_ref[l, 2, :, :] + comm_ref[l, 3, :, :])
            x_cur = total.astype(jnp.bfloat16)

        for k in range(N_DEV):
            @pl.when(my == k)
            def _(k=k):
                out_ref[:, :] = total[k * rows_per:(k + 1) * rows_per, :]

    return pl.pallas_call(
        body,
        out_shape=jax.ShapeDtypeStruct((rows_per, D), jnp.float32),
        in_specs=[pl.BlockSpec(memory_space=pltpu.VMEM)] * 7,
        out_specs=pl.BlockSpec(memory_space=pltpu.VMEM),
        scratch_shapes=[
            pltpu.VMEM((N_LAYERS, N_DEV, B, D), jnp.float32),
            pltpu.SemaphoreType.DMA((N_LAYERS, N_DEV - 1)),
            pltpu.SemaphoreType.DMA((N_LAYERS, N_DEV - 1)),
        ],
        compiler_params=pltpu.CompilerParams(collective_id=0),
    )(x, Win0, Wout0, Win1, Wout1, Win2, Wout2)


# device time: 18504 ns/iter; 1.1489x vs baseline; 1.1489x over previous
import jax
import jax.numpy as jnp
from jax import lax
from jax.experimental import pallas as pl
from jax.experimental.pallas import tpu as pltpu

N_DEV = 4
N_LAYERS = 3


def kernel(x, Win0, Wout0, Win1, Wout1, Win2, Wout2):
    B, D = x.shape
    rows_per = B // N_DEV

    def body(x_ref, win0_ref, wout0_ref, win1_ref, wout1_ref, win2_ref,
             wout2_ref, out_ref, comm_ref, rs_ref, send_sems, recv_sems):
        my = lax.axis_index("i")

        barrier_sem = pltpu.get_barrier_semaphore()
        for d in range(1, N_DEV):
            pl.semaphore_signal(
                barrier_sem, inc=1,
                device_id=((my + d) % N_DEV,),
                device_id_type=pl.DeviceIdType.MESH,
            )
        pl.semaphore_wait(barrier_sem, N_DEV - 1)

        wins = [win0_ref, win1_ref, win2_ref]
        wouts = [wout0_ref, wout1_ref, wout2_ref]

        pending_sends = []
        x_cur = x_ref[:, :].astype(jnp.bfloat16)
        for l in range(N_LAYERS - 1):
            h = jnp.dot(x_cur, wins[l][:, :].astype(jnp.bfloat16),
                        preferred_element_type=jnp.float32)
            h = jnp.maximum(h, 0.0).astype(jnp.bfloat16)
            partial = jnp.dot(h, wouts[l][:, :].astype(jnp.bfloat16),
                              preferred_element_type=jnp.float32
                              ).astype(jnp.bfloat16)
            comm_ref[l, 0, :, :] = partial
            rdmas = []
            for d in (2, 1, 3):
                rdma = pltpu.make_async_remote_copy(
                    src_ref=comm_ref.at[l, 0],
                    dst_ref=comm_ref.at[l, d],
                    send_sem=send_sems.at[l, d - 1],
                    recv_sem=recv_sems.at[l, d - 1],
                    device_id=((my + d) % N_DEV,),
                    device_id_type=pl.DeviceIdType.MESH,
                )
                rdma.start()
                rdmas.append(rdma)
            for rdma in rdmas:
                rdma.wait_recv()
            pending_sends += rdmas
            total = (comm_ref[l, 0, :, :].astype(jnp.float32)
                     + comm_ref[l, 1, :, :].astype(jnp.float32)
                     + comm_ref[l, 2, :, :].astype(jnp.float32)
                     + comm_ref[l, 3, :, :].astype(jnp.float32))
            x_cur = total.astype(jnp.bfloat16)

        l = N_LAYERS - 1
        h = jnp.dot(x_cur, wins[l][:, :].astype(jnp.bfloat16),
                    preferred_element_type=jnp.float32)
        h = jnp.maximum(h, 0.0).astype(jnp.bfloat16)
        partial = jnp.dot(h, wouts[l][:, :].astype(jnp.bfloat16),
                          preferred_element_type=jnp.float32
                          ).astype(jnp.bfloat16)
        rs_ref[0, :, :, :] = partial.reshape(N_DEV, rows_per, D)
        rdmas = []
        for d in (2, 1, 3):
            for k in range(N_DEV):
                @pl.when(my == (k - d) % N_DEV)
                def _(k=k, d=d):
                    rdma = pltpu.make_async_remote_copy(
                        src_ref=rs_ref.at[0, k],
                        dst_ref=rs_ref.at[d, 0],
                        send_sem=send_sems.at[l, d - 1],
                        recv_sem=recv_sems.at[l, d - 1],
                        device_id=(k,),
                        device_id_type=pl.DeviceIdType.MESH,
                    )
                    rdma.start()
        for d in (2, 1, 3):
            rdma = pltpu.make_async_remote_copy(
                src_ref=rs_ref.at[0, 0],
                dst_ref=rs_ref.at[d, 0],
                send_sem=send_sems.at[l, d - 1],
                recv_sem=recv_sems.at[l, d - 1],
                device_id=((my + d) % N_DEV,),
                device_id_type=pl.DeviceIdType.MESH,
            )
            rdma.wait_recv()
            pending_sends.append(rdma)

        for k in range(N_DEV):
            @pl.when(my == k)
            def _(k=k):
                out_ref[:, :] = (rs_ref[0, k, :, :].astype(jnp.float32)
                                 + rs_ref[1, 0, :, :].astype(jnp.float32)
                                 + rs_ref[2, 0, :, :].astype(jnp.float32)
                                 + rs_ref[3, 0, :, :].astype(jnp.float32))

        for rdma in pending_sends:
            rdma.wait_send()

    return pl.pallas_call(
        body,
        out_shape=jax.ShapeDtypeStruct((rows_per, D), jnp.float32),
        in_specs=[pl.BlockSpec(memory_space=pltpu.VMEM)] * 7,
        out_specs=pl.BlockSpec(memory_space=pltpu.VMEM),
        scratch_shapes=[
            pltpu.VMEM((N_LAYERS - 1, N_DEV, B, D), jnp.bfloat16),
            pltpu.VMEM((N_DEV, N_DEV, rows_per, D), jnp.bfloat16),
            pltpu.SemaphoreType.DMA((N_LAYERS, N_DEV - 1)),
            pltpu.SemaphoreType.DMA((N_LAYERS, N_DEV - 1)),
        ],
        compiler_params=pltpu.CompilerParams(collective_id=0),
    )(x, Win0, Wout0, Win1, Wout1, Win2, Wout2)
